# Initial kernel scaffold; baseline (speedup 1.0000x reference)
import jax
import jax.numpy as jnp
from jax.experimental import pallas as pl


def kernel(x, edge_index, W, b):
    raise NotImplementedError("write your pallas kernel here")


# speedup vs baseline: 2.5466x; 2.5466x over previous
import functools

import jax
import jax.numpy as jnp
from jax import lax
from jax.experimental import pallas as pl
from jax.experimental.pallas import tpu as pltpu
from jax.experimental.pallas import tpu_sc as plsc

NC = 1
NS = 16
SUPER = 1024
CHUNK = 128


def _sc_segment_sum(n_pad, e_pad, d, x_pad, src2, dst2, zidx, zc, onesb):
    tile_e = e_pad // (NC * NS)
    n_super = tile_e // SUPER
    rows_per_tile = n_pad // NS
    n_io = rows_per_tile // CHUNK
    mesh = plsc.VectorSubcoreMesh(
        core_axis_name="c", subcore_axis_name="s", num_cores=NC, num_subcores=NS
    )

    @functools.partial(
        pl.kernel,
        out_type=[
            jax.ShapeDtypeStruct((n_pad, d), jnp.float32),
            jax.ShapeDtypeStruct((n_pad, d), jnp.float32),
        ],
        mesh=mesh,
        scratch_types=[
            pltpu.VMEM((SUPER // 128, 128), jnp.int32),
            pltpu.VMEM((SUPER // 128, 128), jnp.int32),
            pltpu.VMEM((CHUNK, d), jnp.float32),
            pltpu.VMEM((n_pad // (NS * 128), 128), jnp.int32),
            pltpu.VMEM_SHARED((n_pad, d), jnp.float32),
            pltpu.SemaphoreType.DMA,
        ],
    )
    def seg_kernel(x_hbm, src_hbm, dst_hbm, zidx_hbm, zc_hbm, ones_hbm,
                   psum_hbm, pcnt_hbm,
                   idx_src, idx_dst, rows, zidx_v, acc, sem):
        c = lax.axis_index("c")
        s = lax.axis_index("s")
        wid = c * NS + s

        r0 = pl.multiple_of(s * rows_per_tile, 8)
        pltpu.sync_copy(zc_hbm, rows)
        pltpu.sync_copy(zidx_hbm.at[s], zidx_v)
        for t in range(n_io):
            pltpu.sync_copy(rows, acc.at[zidx_v.at[t]])
        row0 = pl.multiple_of(wid * (tile_e // 128), 8)
        plsc.subcore_barrier()

        def pass1(g, carry):
            srow = row0 + g * (SUPER // 128)
            pltpu.sync_copy(src_hbm.at[pl.ds(srow, SUPER // 128)], idx_src)
            pltpu.sync_copy(dst_hbm.at[pl.ds(srow, SUPER // 128)], idx_dst)
            for j in range(SUPER // 128):
                pltpu.async_copy(x_hbm.at[idx_src.at[j]], rows, sem).wait()
                pltpu.sync_copy(rows, acc.at[idx_dst.at[j]], add=True)
            return carry

        lax.fori_loop(0, n_super, pass1, 0)
        plsc.subcore_barrier()

        for t in range(n_io):
            rt = r0 + t * CHUNK
            pltpu.async_copy(acc.at[zidx_v.at[t]], rows, sem).wait()
            pltpu.sync_copy(rows, psum_hbm.at[pl.ds(rt, CHUNK)])
        pltpu.sync_copy(zc_hbm, rows)
        for t in range(n_io):
            pltpu.sync_copy(rows, acc.at[zidx_v.at[t]])
        pltpu.sync_copy(ones_hbm, rows)
        plsc.subcore_barrier()

        def pass2(g, carry):
            srow = row0 + g * (SUPER // 128)
            pltpu.sync_copy(dst_hbm.at[pl.ds(srow, SUPER // 128)], idx_dst)
            for j in range(SUPER // 128):
                pltpu.sync_copy(rows, acc.at[idx_dst.at[j]], add=True)
            return carry

        lax.fori_loop(0, n_super, pass2, 0)
        plsc.subcore_barrier()

        for t in range(n_io):
            rt = r0 + t * CHUNK
            pltpu.async_copy(acc.at[zidx_v.at[t]], rows, sem).wait()
            pltpu.sync_copy(rows, pcnt_hbm.at[pl.ds(rt, CHUNK)])

    return seg_kernel(x_pad, src2, dst2, zidx, zc, onesb)


def _tc_bundle(x_ref, psum_ref, pcnt_ref, w1_ref, w2_ref, b_ref, out_ref):
    cnt = pcnt_ref[:, :1]
    c = psum_ref[...] * (1.0 / jnp.maximum(cnt, 1.0))
    z = (
        jnp.dot(x_ref[...], w1_ref[...], preferred_element_type=jnp.float32)
        + jnp.dot(c, w2_ref[...], preferred_element_type=jnp.float32)
        + b_ref[...]
    )
    norm = jnp.sqrt(jnp.sum(z * z, axis=1, keepdims=True))
    out_ref[...] = jnp.maximum(z / jnp.maximum(norm, 1e-12), 0.0)


def kernel(x, edge_index, W, b):
    n, d = x.shape
    e = edge_index.shape[1]
    out_d = W.shape[1]

    block = 512
    n_pad = ((n + block - 1) // block) * block
    tile_e = ((e // (NC * NS) + SUPER - 1) // SUPER) * SUPER
    e_pad = tile_e * NC * NS

    x_pad = jnp.pad(x, ((0, n_pad - n), (0, 0)))
    pad_e = e_pad - e
    src = jnp.concatenate([edge_index[0], jnp.zeros((pad_e,), jnp.int32)])
    src2 = src.reshape(e_pad // 128, 128)
    sink = n + (jnp.arange(pad_e, dtype=jnp.int32) % (n_pad - n))
    dst = jnp.concatenate([edge_index[1], sink])
    dst2 = dst.reshape(e_pad // 128, 128)

    zidx = jnp.arange(n_pad, dtype=jnp.int32).reshape(NS, n_pad // (NS * 128), 128)
    zc = jnp.zeros((CHUNK, d), jnp.float32)
    onesb = jnp.ones((CHUNK, d), jnp.float32)

    psum, pcnt = _sc_segment_sum(n_pad, e_pad, d, x_pad, src2, dst2, zidx, zc, onesb)

    grid = n_pad // block
    out = pl.pallas_call(
        _tc_bundle,
        grid=(grid,),
        in_specs=[
            pl.BlockSpec((block, d), lambda i: (i, 0)),
            pl.BlockSpec((block, d), lambda i: (i, 0)),
            pl.BlockSpec((block, d), lambda i: (i, 0)),
            pl.BlockSpec((d, out_d), lambda i: (0, 0)),
            pl.BlockSpec((d, out_d), lambda i: (0, 0)),
            pl.BlockSpec((1, out_d), lambda i: (0, 0)),
        ],
        out_specs=pl.BlockSpec((block, out_d), lambda i: (i, 0)),
        out_shape=jax.ShapeDtypeStruct((n_pad, out_d), jnp.float32),
    )(x_pad, psum, pcnt, W[:d], W[d:], b.reshape(1, out_d))
    return out[:n]


# speedup vs baseline: 2.8048x; 1.1014x over previous
import functools

import jax
import jax.numpy as jnp
from jax import lax
from jax.experimental import pallas as pl
from jax.experimental.pallas import tpu as pltpu
from jax.experimental.pallas import tpu_sc as plsc

NC = 1
NS = 16
SUPER = 1024
CHUNK = 128


def _sc_segment_sum(n_pad, e_pad, d, x_pad, src2, dst2, zidx, zc, onesb):
    tile_e = e_pad // (NC * NS)
    n_super = tile_e // SUPER
    rows_per_tile = n_pad // NS
    n_io = rows_per_tile // CHUNK
    mesh = plsc.VectorSubcoreMesh(
        core_axis_name="c", subcore_axis_name="s", num_cores=NC, num_subcores=NS
    )

    @functools.partial(
        pl.kernel,
        out_type=[
            jax.ShapeDtypeStruct((n_pad, d), jnp.float32),
            jax.ShapeDtypeStruct((n_pad, d), jnp.float32),
        ],
        mesh=mesh,
        scratch_types=[
            pltpu.VMEM((SUPER // 128, 128), jnp.int32),
            pltpu.VMEM((SUPER // 128, 128), jnp.int32),
            pltpu.VMEM((CHUNK, d), jnp.float32),
            pltpu.VMEM((CHUNK, d), jnp.float32),
            pltpu.VMEM((n_pad // (NS * 128), 128), jnp.int32),
            pltpu.VMEM_SHARED((n_pad, d), jnp.float32),
            pltpu.SemaphoreType.DMA,
            pltpu.SemaphoreType.DMA,
            pltpu.SemaphoreType.DMA,
        ],
    )
    def seg_kernel(x_hbm, src_hbm, dst_hbm, zidx_hbm, zc_hbm, ones_hbm,
                   psum_hbm, pcnt_hbm,
                   idx_src, idx_dst, rows, rows2, zidx_v, acc, sem, sem2, sem3):
        c = lax.axis_index("c")
        s = lax.axis_index("s")
        wid = c * NS + s

        r0 = pl.multiple_of(s * rows_per_tile, 8)
        pltpu.sync_copy(zc_hbm, rows)
        pltpu.sync_copy(zidx_hbm.at[s], zidx_v)
        for t in range(n_io):
            pltpu.sync_copy(rows, acc.at[zidx_v.at[t]])
        row0 = pl.multiple_of(wid * (tile_e // 128), 8)
        plsc.subcore_barrier()

        bufs = (rows, rows2)
        sems = (sem, sem2)

        def pass1(g, carry):
            srow = row0 + g * (SUPER // 128)
            pltpu.sync_copy(src_hbm.at[pl.ds(srow, SUPER // 128)], idx_src)
            pltpu.sync_copy(dst_hbm.at[pl.ds(srow, SUPER // 128)], idx_dst)
            n_j = SUPER // 128
            pend = pltpu.async_copy(x_hbm.at[idx_src.at[0]], bufs[0], sems[0])
            for j in range(n_j):
                pend.wait()
                if j + 1 < n_j:
                    pend = pltpu.async_copy(
                        x_hbm.at[idx_src.at[j + 1]], bufs[(j + 1) % 2],
                        sems[(j + 1) % 2])
                pltpu.sync_copy(bufs[j % 2], acc.at[idx_dst.at[j]], add=True)
            return carry

        lax.fori_loop(0, n_super, pass1, 0)
        plsc.subcore_barrier()

        for t in range(n_io):
            rt = r0 + t * CHUNK
            pltpu.async_copy(acc.at[zidx_v.at[t]], rows, sem).wait()
            pltpu.sync_copy(rows, psum_hbm.at[pl.ds(rt, CHUNK)])
        pltpu.sync_copy(zc_hbm, rows)
        for t in range(n_io):
            pltpu.sync_copy(rows, acc.at[zidx_v.at[t]])
        pltpu.sync_copy(ones_hbm, rows)
        plsc.subcore_barrier()

        def pass2(g, carry):
            srow = row0 + g * (SUPER // 128)
            pltpu.sync_copy(dst_hbm.at[pl.ds(srow, SUPER // 128)], idx_dst)
            pends = [
                pltpu.async_copy(rows, acc.at[idx_dst.at[j]], sem3, add=True)
                for j in range(SUPER // 128)
            ]
            for p in pends:
                p.wait()
            return carry

        lax.fori_loop(0, n_super, pass2, 0)
        plsc.subcore_barrier()

        for t in range(n_io):
            rt = r0 + t * CHUNK
            pltpu.async_copy(acc.at[zidx_v.at[t]], rows, sem).wait()
            pltpu.sync_copy(rows, pcnt_hbm.at[pl.ds(rt, CHUNK)])

    return seg_kernel(x_pad, src2, dst2, zidx, zc, onesb)


def _tc_bundle(x_ref, psum_ref, pcnt_ref, w1_ref, w2_ref, b_ref, out_ref):
    cnt = pcnt_ref[:, :1]
    c = psum_ref[...] * (1.0 / jnp.maximum(cnt, 1.0))
    z = (
        jnp.dot(x_ref[...], w1_ref[...], preferred_element_type=jnp.float32)
        + jnp.dot(c, w2_ref[...], preferred_element_type=jnp.float32)
        + b_ref[...]
    )
    norm = jnp.sqrt(jnp.sum(z * z, axis=1, keepdims=True))
    out_ref[...] = jnp.maximum(z / jnp.maximum(norm, 1e-12), 0.0)


def kernel(x, edge_index, W, b):
    n, d = x.shape
    e = edge_index.shape[1]
    out_d = W.shape[1]

    block = 512
    n_pad = ((n + block - 1) // block) * block
    tile_e = ((e // (NC * NS) + SUPER - 1) // SUPER) * SUPER
    e_pad = tile_e * NC * NS

    x_pad = jnp.pad(x, ((0, n_pad - n), (0, 0)))
    pad_e = e_pad - e
    src = jnp.concatenate([edge_index[0], jnp.zeros((pad_e,), jnp.int32)])
    src2 = src.reshape(e_pad // 128, 128)
    sink = n + (jnp.arange(pad_e, dtype=jnp.int32) % (n_pad - n))
    dst = jnp.concatenate([edge_index[1], sink])
    dst2 = dst.reshape(e_pad // 128, 128)

    zidx = jnp.arange(n_pad, dtype=jnp.int32).reshape(NS, n_pad // (NS * 128), 128)
    zc = jnp.zeros((CHUNK, d), jnp.float32)
    onesb = jnp.ones((CHUNK, d), jnp.float32)

    psum, pcnt = _sc_segment_sum(n_pad, e_pad, d, x_pad, src2, dst2, zidx, zc, onesb)

    grid = n_pad // block
    out = pl.pallas_call(
        _tc_bundle,
        grid=(grid,),
        in_specs=[
            pl.BlockSpec((block, d), lambda i: (i, 0)),
            pl.BlockSpec((block, d), lambda i: (i, 0)),
            pl.BlockSpec((block, d), lambda i: (i, 0)),
            pl.BlockSpec((d, out_d), lambda i: (0, 0)),
            pl.BlockSpec((d, out_d), lambda i: (0, 0)),
            pl.BlockSpec((1, out_d), lambda i: (0, 0)),
        ],
        out_specs=pl.BlockSpec((block, out_d), lambda i: (i, 0)),
        out_shape=jax.ShapeDtypeStruct((n_pad, out_d), jnp.float32),
    )(x_pad, psum, pcnt, W[:d], W[d:], b.reshape(1, out_d))
    return out[:n]


# speedup vs baseline: 3.3170x; 1.1826x over previous
import functools

import jax
import jax.numpy as jnp
from jax import lax
from jax.experimental import pallas as pl
from jax.experimental.pallas import tpu as pltpu
from jax.experimental.pallas import tpu_sc as plsc

NC = 1
NS = 16
SUPER = 1024
CHUNK = 128
LANES = 16


def _sc_segment_sum(n_pad, e_pad, d, x_pad, src2, dst2, zidx, zc, zcnt):
    tile_e = e_pad // (NC * NS)
    n_super = tile_e // SUPER
    rows_per_tile = n_pad // NS
    n_io = rows_per_tile // CHUNK
    mesh = plsc.VectorSubcoreMesh(
        core_axis_name="c", subcore_axis_name="s", num_cores=NC, num_subcores=NS
    )

    @functools.partial(
        pl.kernel,
        out_type=[
            jax.ShapeDtypeStruct((n_pad, d), jnp.float32),
            jax.ShapeDtypeStruct((NS, n_pad), jnp.float32),
        ],
        mesh=mesh,
        compiler_params=pltpu.CompilerParams(needs_layout_passes=False),
        scratch_types=[
            pltpu.VMEM((SUPER // 128, 128), jnp.int32),
            pltpu.VMEM((SUPER // 128, 128), jnp.int32),
            pltpu.VMEM((CHUNK, d), jnp.float32),
            pltpu.VMEM((CHUNK, d), jnp.float32),
            pltpu.VMEM((n_pad,), jnp.float32),
            pltpu.VMEM((n_pad // (NS * 128), 128), jnp.int32),
            pltpu.VMEM_SHARED((n_pad, d), jnp.float32),
            pltpu.SemaphoreType.DMA,
            pltpu.SemaphoreType.DMA,
        ],
    )
    def seg_kernel(x_hbm, src_hbm, dst_hbm, zidx_hbm, zc_hbm, zcnt_hbm,
                   psum_hbm, pcnt_hbm,
                   idx_src, idx_dst, rows, rows2, cnt_v, zidx_v, acc, sem, sem2):
        c = lax.axis_index("c")
        s = lax.axis_index("s")
        wid = c * NS + s

        r0 = pl.multiple_of(s * rows_per_tile, 8)
        pltpu.sync_copy(zc_hbm, rows)
        pltpu.sync_copy(zcnt_hbm, cnt_v)
        pltpu.sync_copy(zidx_hbm.at[s], zidx_v)
        for t in range(n_io):
            pltpu.sync_copy(rows, acc.at[zidx_v.at[t]])
        row0 = pl.multiple_of(wid * (tile_e // 128), 8)
        plsc.subcore_barrier()

        ones16 = jnp.ones((LANES,), jnp.float32)
        bufs = (rows, rows2)
        sems = (sem, sem2)
        n_j = SUPER // 128

        def edge_pass(g, carry):
            srow = row0 + g * n_j
            pltpu.sync_copy(src_hbm.at[pl.ds(srow, n_j)], idx_src)
            pltpu.sync_copy(dst_hbm.at[pl.ds(srow, n_j)], idx_dst)
            pend = pltpu.async_copy(x_hbm.at[idx_src.at[0]], bufs[0], sems[0])
            for j in range(n_j):
                pend.wait()
                if j + 1 < n_j:
                    pend = pltpu.async_copy(
                        x_hbm.at[idx_src.at[j + 1]], bufs[(j + 1) % 2],
                        sems[(j + 1) % 2])
                pltpu.sync_copy(bufs[j % 2], acc.at[idx_dst.at[j]], add=True)
                for k in range(128 // LANES):
                    idx16 = idx_dst[j, pl.ds(k * LANES, LANES)]
                    plsc.addupdate_scatter(cnt_v, [idx16], ones16)
            return carry

        lax.fori_loop(0, n_super, edge_pass, 0)
        plsc.subcore_barrier()

        for t in range(n_io):
            rt = r0 + t * CHUNK
            pltpu.async_copy(acc.at[zidx_v.at[t]], rows, sem).wait()
            pltpu.sync_copy(rows, psum_hbm.at[pl.ds(rt, CHUNK)])
        pltpu.sync_copy(cnt_v, pcnt_hbm.at[s])

    return seg_kernel(x_pad, src2, dst2, zidx, zc, zcnt)


def _tc_bundle(x_ref, psum_ref, pcnt_ref, w1_ref, w2_ref, b_ref, out_ref):
    cnt = lax.dot_general(
        pcnt_ref[...], jnp.ones((NS, 1), jnp.float32),
        (((0,), (0,)), ((), ())), preferred_element_type=jnp.float32)
    c = psum_ref[...] * (1.0 / jnp.maximum(cnt, 1.0))
    z = (
        jnp.dot(x_ref[...], w1_ref[...], preferred_element_type=jnp.float32)
        + jnp.dot(c, w2_ref[...], preferred_element_type=jnp.float32)
        + b_ref[...]
    )
    norm = jnp.sqrt(jnp.sum(z * z, axis=1, keepdims=True))
    out_ref[...] = jnp.maximum(z / jnp.maximum(norm, 1e-12), 0.0)


def kernel(x, edge_index, W, b):
    n, d = x.shape
    e = edge_index.shape[1]
    out_d = W.shape[1]

    block = 512
    n_pad = ((n + block - 1) // block) * block
    tile_e = ((e // (NC * NS) + SUPER - 1) // SUPER) * SUPER
    e_pad = tile_e * NC * NS

    x_pad = jnp.pad(x, ((0, n_pad - n), (0, 0)))
    pad_e = e_pad - e
    src = jnp.concatenate([edge_index[0], jnp.zeros((pad_e,), jnp.int32)])
    src2 = src.reshape(e_pad // 128, 128)
    sink = n + (jnp.arange(pad_e, dtype=jnp.int32) % (n_pad - n))
    dst = jnp.concatenate([edge_index[1], sink])
    dst2 = dst.reshape(e_pad // 128, 128)

    zidx = jnp.arange(n_pad, dtype=jnp.int32).reshape(NS, n_pad // (NS * 128), 128)
    zc = jnp.zeros((CHUNK, d), jnp.float32)
    zcnt = jnp.zeros((n_pad,), jnp.float32)

    psum, pcnt = _sc_segment_sum(n_pad, e_pad, d, x_pad, src2, dst2, zidx, zc, zcnt)

    grid = n_pad // block
    out = pl.pallas_call(
        _tc_bundle,
        grid=(grid,),
        in_specs=[
            pl.BlockSpec((block, d), lambda i: (i, 0)),
            pl.BlockSpec((block, d), lambda i: (i, 0)),
            pl.BlockSpec((NS, block), lambda i: (0, i)),
            pl.BlockSpec((d, out_d), lambda i: (0, 0)),
            pl.BlockSpec((d, out_d), lambda i: (0, 0)),
            pl.BlockSpec((1, out_d), lambda i: (0, 0)),
        ],
        out_specs=pl.BlockSpec((block, out_d), lambda i: (i, 0)),
        out_shape=jax.ShapeDtypeStruct((n_pad, out_d), jnp.float32),
    )(x_pad, psum, pcnt, W[:d], W[d:], b.reshape(1, out_d))
    return out[:n]


# speedup vs baseline: 3.3200x; 1.0009x over previous
import functools

import jax
import jax.numpy as jnp
from jax import lax
from jax.experimental import pallas as pl
from jax.experimental.pallas import tpu as pltpu
from jax.experimental.pallas import tpu_sc as plsc

NC = 1
NS = 16
SUPER = 1024
CHUNK = 128
LANES = 16


def _sc_segment_sum(n_pad, e_pad, d, x_pad, src2, dst2, zidx, zc, zcnt):
    tile_e = e_pad // (NC * NS)
    n_super = tile_e // SUPER
    rows_per_tile = n_pad // NS
    n_io = rows_per_tile // CHUNK
    mesh = plsc.VectorSubcoreMesh(
        core_axis_name="c", subcore_axis_name="s", num_cores=NC, num_subcores=NS
    )

    @functools.partial(
        pl.kernel,
        out_type=[
            jax.ShapeDtypeStruct((n_pad, d), jnp.float32),
            jax.ShapeDtypeStruct((NS, n_pad), jnp.float32),
        ],
        mesh=mesh,
        compiler_params=pltpu.CompilerParams(needs_layout_passes=False),
        scratch_types=[
            pltpu.VMEM((SUPER // 128, 128), jnp.int32),
            pltpu.VMEM((SUPER // 128, 128), jnp.int32),
            pltpu.VMEM((CHUNK, d), jnp.float32),
            pltpu.VMEM((CHUNK, d), jnp.float32),
            pltpu.VMEM((n_pad,), jnp.float32),
            pltpu.VMEM((n_pad // (NS * 128), 128), jnp.int32),
            pltpu.VMEM_SHARED((n_pad, d), jnp.float32),
            pltpu.SemaphoreType.DMA,
            pltpu.SemaphoreType.DMA,
            pltpu.SemaphoreType.DMA,
            pltpu.SemaphoreType.DMA,
        ],
    )
    def seg_kernel(x_hbm, src_hbm, dst_hbm, zidx_hbm, zc_hbm, zcnt_hbm,
                   psum_hbm, pcnt_hbm,
                   idx_src, idx_dst, rows, rows2, cnt_v, zidx_v, acc,
                   sem, sem2, ssem, ssem2):
        c = lax.axis_index("c")
        s = lax.axis_index("s")
        wid = c * NS + s

        r0 = pl.multiple_of(s * rows_per_tile, 8)
        pltpu.sync_copy(zc_hbm, rows)
        pltpu.sync_copy(zcnt_hbm, cnt_v)
        pltpu.sync_copy(zidx_hbm.at[s], zidx_v)
        for t in range(n_io):
            pltpu.sync_copy(rows, acc.at[zidx_v.at[t]])
        row0 = pl.multiple_of(wid * (tile_e // 128), 8)
        plsc.subcore_barrier()

        ones16 = jnp.ones((LANES,), jnp.float32)
        bufs = (rows, rows2)
        sems = (sem, sem2)
        ssems = (ssem, ssem2)
        n_j = SUPER // 128

        def edge_pass(g, carry):
            srow = row0 + g * n_j
            pltpu.sync_copy(src_hbm.at[pl.ds(srow, n_j)], idx_src)
            pltpu.sync_copy(dst_hbm.at[pl.ds(srow, n_j)], idx_dst)
            pend_g = [None, None]
            pend_s = [None, None]
            pend_g[0] = pltpu.async_copy(x_hbm.at[idx_src.at[0]], bufs[0], sems[0])
            for j in range(n_j):
                b = j % 2
                nb = (j + 1) % 2
                pend_g[b].wait()
                if j + 1 < n_j:
                    if pend_s[nb] is not None:
                        pend_s[nb].wait()
                    pend_g[nb] = pltpu.async_copy(
                        x_hbm.at[idx_src.at[j + 1]], bufs[nb], sems[nb])
                pend_s[b] = pltpu.async_copy(
                    bufs[b], acc.at[idx_dst.at[j]], ssems[b], add=True)
                for k in range(128 // LANES):
                    idx16 = idx_dst[j, pl.ds(k * LANES, LANES)]
                    plsc.addupdate_scatter(cnt_v, [idx16], ones16)
            for p in pend_s:
                if p is not None:
                    p.wait()
            return carry

        lax.fori_loop(0, n_super, edge_pass, 0)
        plsc.subcore_barrier()

        for t in range(n_io):
            rt = r0 + t * CHUNK
            pltpu.async_copy(acc.at[zidx_v.at[t]], rows, sem).wait()
            pltpu.sync_copy(rows, psum_hbm.at[pl.ds(rt, CHUNK)])
        pltpu.sync_copy(cnt_v, pcnt_hbm.at[s])

    return seg_kernel(x_pad, src2, dst2, zidx, zc, zcnt)


def _tc_bundle(x_ref, psum_ref, pcnt_ref, w1_ref, w2_ref, b_ref, out_ref):
    cnt = lax.dot_general(
        pcnt_ref[...], jnp.ones((NS, 1), jnp.float32),
        (((0,), (0,)), ((), ())), preferred_element_type=jnp.float32)
    c = psum_ref[...] * (1.0 / jnp.maximum(cnt, 1.0))
    z = (
        jnp.dot(x_ref[...], w1_ref[...], preferred_element_type=jnp.float32)
        + jnp.dot(c, w2_ref[...], preferred_element_type=jnp.float32)
        + b_ref[...]
    )
    norm = jnp.sqrt(jnp.sum(z * z, axis=1, keepdims=True))
    out_ref[...] = jnp.maximum(z / jnp.maximum(norm, 1e-12), 0.0)


def kernel(x, edge_index, W, b):
    n, d = x.shape
    e = edge_index.shape[1]
    out_d = W.shape[1]

    block = 512
    n_pad = ((n + block - 1) // block) * block
    tile_e = ((e // (NC * NS) + SUPER - 1) // SUPER) * SUPER
    e_pad = tile_e * NC * NS

    x_pad = jnp.pad(x, ((0, n_pad - n), (0, 0)))
    pad_e = e_pad - e
    src = jnp.concatenate([edge_index[0], jnp.zeros((pad_e,), jnp.int32)])
    src2 = src.reshape(e_pad // 128, 128)
    sink = n + (jnp.arange(pad_e, dtype=jnp.int32) % (n_pad - n))
    dst = jnp.concatenate([edge_index[1], sink])
    dst2 = dst.reshape(e_pad // 128, 128)

    zidx = jnp.arange(n_pad, dtype=jnp.int32).reshape(NS, n_pad // (NS * 128), 128)
    zc = jnp.zeros((CHUNK, d), jnp.float32)
    zcnt = jnp.zeros((n_pad,), jnp.float32)

    psum, pcnt = _sc_segment_sum(n_pad, e_pad, d, x_pad, src2, dst2, zidx, zc, zcnt)

    grid = n_pad // block
    out = pl.pallas_call(
        _tc_bundle,
        grid=(grid,),
        in_specs=[
            pl.BlockSpec((block, d), lambda i: (i, 0)),
            pl.BlockSpec((block, d), lambda i: (i, 0)),
            pl.BlockSpec((NS, block), lambda i: (0, i)),
            pl.BlockSpec((d, out_d), lambda i: (0, 0)),
            pl.BlockSpec((d, out_d), lambda i: (0, 0)),
            pl.BlockSpec((1, out_d), lambda i: (0, 0)),
        ],
        out_specs=pl.BlockSpec((block, out_d), lambda i: (i, 0)),
        out_shape=jax.ShapeDtypeStruct((n_pad, out_d), jnp.float32),
    )(x_pad, psum, pcnt, W[:d], W[d:], b.reshape(1, out_d))
    return out[:n]


# speedup vs baseline: 3.3922x; 1.0218x over previous
import functools

import jax
import jax.numpy as jnp
from jax import lax
from jax.experimental import pallas as pl
from jax.experimental.pallas import tpu as pltpu
from jax.experimental.pallas import tpu_sc as plsc

NC = 1
NS = 16
SUPER = 2048
CHUNK = 128
LANES = 16


def _sc_segment_sum(n_pad, e_pad, d, x_pad, src2, dst2, zidx, zc, zcnt):
    tile_e = e_pad // (NC * NS)
    n_super = tile_e // SUPER
    rows_per_tile = n_pad // NS
    n_io = rows_per_tile // CHUNK
    mesh = plsc.VectorSubcoreMesh(
        core_axis_name="c", subcore_axis_name="s", num_cores=NC, num_subcores=NS
    )

    @functools.partial(
        pl.kernel,
        out_type=[
            jax.ShapeDtypeStruct((n_pad, d), jnp.float32),
            jax.ShapeDtypeStruct((NS, n_pad), jnp.float32),
        ],
        mesh=mesh,
        compiler_params=pltpu.CompilerParams(needs_layout_passes=False),
        scratch_types=[
            pltpu.VMEM((SUPER // 128, 128), jnp.int32),
            pltpu.VMEM((SUPER // 128, 128), jnp.int32),
            pltpu.VMEM((CHUNK, d), jnp.float32),
            pltpu.VMEM((CHUNK, d), jnp.float32),
            pltpu.VMEM((n_pad,), jnp.float32),
            pltpu.VMEM((n_pad // (NS * 128), 128), jnp.int32),
            pltpu.VMEM_SHARED((n_pad, d), jnp.float32),
            pltpu.SemaphoreType.DMA,
            pltpu.SemaphoreType.DMA,
            pltpu.SemaphoreType.DMA,
            pltpu.SemaphoreType.DMA,
        ],
    )
    def seg_kernel(x_hbm, src_hbm, dst_hbm, zidx_hbm, zc_hbm, zcnt_hbm,
                   psum_hbm, pcnt_hbm,
                   idx_src, idx_dst, rows, rows2, cnt_v, zidx_v, acc,
                   sem, sem2, ssem, ssem2):
        c = lax.axis_index("c")
        s = lax.axis_index("s")
        wid = c * NS + s

        r0 = pl.multiple_of(s * rows_per_tile, 8)
        pltpu.sync_copy(zc_hbm, rows)
        pltpu.sync_copy(zcnt_hbm, cnt_v)
        pltpu.sync_copy(zidx_hbm.at[s], zidx_v)
        for t in range(n_io):
            pltpu.sync_copy(rows, acc.at[zidx_v.at[t]])
        row0 = pl.multiple_of(wid * (tile_e // 128), 8)
        plsc.subcore_barrier()

        ones16 = jnp.ones((LANES,), jnp.float32)
        bufs = (rows, rows2)
        sems = (sem, sem2)
        ssems = (ssem, ssem2)
        n_j = SUPER // 128

        def edge_pass(g, carry):
            srow = row0 + g * n_j
            pltpu.sync_copy(src_hbm.at[pl.ds(srow, n_j)], idx_src)
            pltpu.sync_copy(dst_hbm.at[pl.ds(srow, n_j)], idx_dst)
            pend_g = [None, None]
            pend_s = [None, None]
            pend_g[0] = pltpu.async_copy(x_hbm.at[idx_src.at[0]], bufs[0], sems[0])
            for j in range(n_j):
                b = j % 2
                nb = (j + 1) % 2
                pend_g[b].wait()
                if j + 1 < n_j:
                    if pend_s[nb] is not None:
                        pend_s[nb].wait()
                    pend_g[nb] = pltpu.async_copy(
                        x_hbm.at[idx_src.at[j + 1]], bufs[nb], sems[nb])
                pend_s[b] = pltpu.async_copy(
                    bufs[b], acc.at[idx_dst.at[j]], ssems[b], add=True)
                for k in range(128 // LANES):
                    idx16 = idx_dst[j, pl.ds(k * LANES, LANES)]
                    plsc.addupdate_scatter(cnt_v, [idx16], ones16)
            for p in pend_s:
                if p is not None:
                    p.wait()
            return carry

        lax.fori_loop(0, n_super, edge_pass, 0)
        plsc.subcore_barrier()

        for t in range(n_io):
            rt = r0 + t * CHUNK
            pltpu.async_copy(acc.at[zidx_v.at[t]], rows, sem).wait()
            pltpu.sync_copy(rows, psum_hbm.at[pl.ds(rt, CHUNK)])
        pltpu.sync_copy(cnt_v, pcnt_hbm.at[s])

    return seg_kernel(x_pad, src2, dst2, zidx, zc, zcnt)


def _tc_bundle(x_ref, psum_ref, pcnt_ref, w1_ref, w2_ref, b_ref, out_ref):
    cnt = lax.dot_general(
        pcnt_ref[...], jnp.ones((NS, 1), jnp.float32),
        (((0,), (0,)), ((), ())), preferred_element_type=jnp.float32)
    c = psum_ref[...] * (1.0 / jnp.maximum(cnt, 1.0))
    z = (
        jnp.dot(x_ref[...], w1_ref[...], preferred_element_type=jnp.float32)
        + jnp.dot(c, w2_ref[...], preferred_element_type=jnp.float32)
        + b_ref[...]
    )
    norm = jnp.sqrt(jnp.sum(z * z, axis=1, keepdims=True))
    out_ref[...] = jnp.maximum(z / jnp.maximum(norm, 1e-12), 0.0)


def kernel(x, edge_index, W, b):
    n, d = x.shape
    e = edge_index.shape[1]
    out_d = W.shape[1]

    block = 512
    n_pad = ((n + block - 1) // block) * block
    tile_e = ((e // (NC * NS) + SUPER - 1) // SUPER) * SUPER
    e_pad = tile_e * NC * NS

    x_pad = jnp.pad(x, ((0, n_pad - n), (0, 0)))
    pad_e = e_pad - e
    src = jnp.concatenate([edge_index[0], jnp.zeros((pad_e,), jnp.int32)])
    src2 = src.reshape(e_pad // 128, 128)
    sink = n + (jnp.arange(pad_e, dtype=jnp.int32) % (n_pad - n))
    dst = jnp.concatenate([edge_index[1], sink])
    dst2 = dst.reshape(e_pad // 128, 128)

    zidx = jnp.arange(n_pad, dtype=jnp.int32).reshape(NS, n_pad // (NS * 128), 128)
    zc = jnp.zeros((CHUNK, d), jnp.float32)
    zcnt = jnp.zeros((n_pad,), jnp.float32)

    psum, pcnt = _sc_segment_sum(n_pad, e_pad, d, x_pad, src2, dst2, zidx, zc, zcnt)

    grid = n_pad // block
    out = pl.pallas_call(
        _tc_bundle,
        grid=(grid,),
        in_specs=[
            pl.BlockSpec((block, d), lambda i: (i, 0)),
            pl.BlockSpec((block, d), lambda i: (i, 0)),
            pl.BlockSpec((NS, block), lambda i: (0, i)),
            pl.BlockSpec((d, out_d), lambda i: (0, 0)),
            pl.BlockSpec((d, out_d), lambda i: (0, 0)),
            pl.BlockSpec((1, out_d), lambda i: (0, 0)),
        ],
        out_specs=pl.BlockSpec((block, out_d), lambda i: (i, 0)),
        out_shape=jax.ShapeDtypeStruct((n_pad, out_d), jnp.float32),
    )(x_pad, psum, pcnt, W[:d], W[d:], b.reshape(1, out_d))
    return out[:n]
